# baseline (speedup 1.0000x reference)
import functools

import jax
import jax.numpy as jnp
from jax import lax
from jax.experimental import pallas as pl
from jax.experimental.pallas import tpu as pltpu
from jax.experimental.pallas import tpu_sc as plsc

NUM_TABLES = 26
VOCAB = 100000
DIM = 32
BATCH = 16384

NC = 2
NS = 16
L = 16
NW = NC * NS

PACK = 128 // DIM
PROWS = NUM_TABLES * VOCAB // PACK
TOTAL = NUM_TABLES * BATCH

CHUNK = 512
GATHER = 128
N_GATHER = CHUNK // GATHER
CHUNKS_PER_TABLE = BATCH // CHUNK
N_CHUNKS = TOTAL // (NW * CHUNK)
GROUPS = CHUNK // L


def _sc_lookup(ids_flat, tables_packed):
    mesh = plsc.VectorSubcoreMesh(core_axis_name="c", subcore_axis_name="s")

    @functools.partial(
        pl.kernel,
        mesh=mesh,
        compiler_params=pltpu.CompilerParams(
            use_tc_tiling_on_sc=True, needs_layout_passes=False
        ),
        out_type=jax.ShapeDtypeStruct((TOTAL * DIM,), jnp.float32),
        scratch_types=[
            pltpu.VMEM((CHUNK,), jnp.int32),
            pltpu.VMEM((N_GATHER, GATHER), jnp.int32),
            pltpu.VMEM((CHUNK,), jnp.int32),
            pltpu.VMEM((CHUNK, 128), jnp.float32),
            pltpu.VMEM((CHUNK * DIM,), jnp.float32),
            pltpu.SMEM((CHUNK,), jnp.int32),
            pltpu.SemaphoreType.DMA,
        ],
    )
    def k(ids_hbm, tab_hbm, out_hbm, ids_v, pidx_v, g_v, buf_v, sel_v, off_s, sem):
        wid = lax.axis_index("s") * NC + lax.axis_index("c")
        iota = lax.iota(jnp.int32, L)

        def chunk_body(j, carry):
            c = wid * N_CHUNKS + j
            base = c * CHUNK
            t = c // CHUNKS_PER_TABLE
            off = t * VOCAB
            pltpu.sync_copy(ids_hbm.at[pl.ds(base, CHUNK)], ids_v)

            @plsc.parallel_loop(0, GROUPS, unroll=4)
            def idx_body(v):
                flat = ids_v[pl.ds(v * L, L)] + off
                r = v // (GATHER // L)
                col = (v % (GATHER // L)) * L
                pidx_v[r, pl.ds(col, L)] = lax.shift_right_logical(flat, 2)
                g_v[pl.ds(v * L, L)] = lax.bitwise_and(flat, 3) * DIM

            copies = []
            for q in range(N_GATHER):
                copies.append(
                    pltpu.make_async_copy(
                        tab_hbm.at[pidx_v.at[q]],
                        buf_v.at[pl.ds(q * GATHER, GATHER)],
                        sem,
                    )
                )
                copies[-1].start()
            for cp in copies:
                cp.wait()

            @plsc.parallel_loop(0, GROUPS, unroll=2)
            def sel_body(v):
                rows = v * L + iota
                colb = g_v[pl.ds(v * L, L)]
                wb = rows * DIM
                vals = []
                for d in range(DIM):
                    vals.append(plsc.load_gather(buf_v, [rows, colb + d]))
                for d in range(DIM):
                    plsc.store_scatter(sel_v, [wb + d], vals[d])

            pltpu.sync_copy(sel_v, out_hbm.at[pl.ds(base * DIM, CHUNK * DIM)])
            return carry

        lax.fori_loop(0, N_CHUNKS, chunk_body, 0)

    return k(ids_flat, tables_packed)


def kernel(ids, tables):
    out_flat = _sc_lookup(
        ids.reshape(TOTAL),
        tables.reshape(PROWS, PACK * DIM),
    )
    return out_flat.reshape(NUM_TABLES, BATCH, DIM)


# speedup vs baseline: 1.3440x; 1.3440x over previous
import functools

import jax
import jax.numpy as jnp
from jax import lax
from jax.experimental import pallas as pl
from jax.experimental.pallas import tpu as pltpu
from jax.experimental.pallas import tpu_sc as plsc

NUM_TABLES = 26
VOCAB = 100000
DIM = 32
BATCH = 16384

NC = 2
NS = 16
L = 16
NW = NC * NS

ROWS = NUM_TABLES * VOCAB
TOTAL = NUM_TABLES * BATCH

CHUNK = 1024
GATHER = 128
N_GATHER = CHUNK // GATHER
CHUNKS_PER_TABLE = BATCH // CHUNK
N_CHUNKS = TOTAL // (NW * CHUNK)
GROUPS = CHUNK // L


def _sc_lookup(ids_flat, tables_flat):
    mesh = plsc.VectorSubcoreMesh(core_axis_name="c", subcore_axis_name="s")

    @functools.partial(
        pl.kernel,
        mesh=mesh,
        compiler_params=pltpu.CompilerParams(
            use_tc_tiling_on_sc=False, needs_layout_passes=False
        ),
        out_type=jax.ShapeDtypeStruct((TOTAL, DIM), jnp.float32),
        scratch_types=[
            pltpu.VMEM((CHUNK,), jnp.int32),
            pltpu.VMEM((N_GATHER, GATHER), jnp.int32),
            pltpu.VMEM((CHUNK, DIM), jnp.float32),
            pltpu.SemaphoreType.DMA,
        ],
    )
    def k(ids_hbm, tab_hbm, out_hbm, ids_v, pidx_v, rows_v, sem):
        wid = lax.axis_index("s") * NC + lax.axis_index("c")

        def chunk_body(j, carry):
            c = wid * N_CHUNKS + j
            base = c * CHUNK
            t = c // CHUNKS_PER_TABLE
            off = t * VOCAB
            pltpu.sync_copy(ids_hbm.at[pl.ds(base, CHUNK)], ids_v)

            @plsc.parallel_loop(0, GROUPS, unroll=4)
            def idx_body(v):
                r = v // (GATHER // L)
                col = (v % (GATHER // L)) * L
                pidx_v[r, pl.ds(col, L)] = ids_v[pl.ds(v * L, L)] + off

            copies = []
            for q in range(N_GATHER):
                copies.append(
                    pltpu.make_async_copy(
                        tab_hbm.at[pidx_v.at[q]],
                        rows_v.at[pl.ds(q * GATHER, GATHER)],
                        sem,
                    )
                )
                copies[-1].start()
            for cp in copies:
                cp.wait()

            pltpu.sync_copy(rows_v, out_hbm.at[pl.ds(base, CHUNK)])
            return carry

        lax.fori_loop(0, N_CHUNKS, chunk_body, 0)

    return k(ids_flat, tables_flat)


def kernel(ids, tables):
    out_flat = _sc_lookup(
        ids.reshape(TOTAL),
        tables.reshape(ROWS, DIM),
    )
    return out_flat.reshape(NUM_TABLES, BATCH, DIM)
